# Initial kernel scaffold; baseline (speedup 1.0000x reference)
import jax
import jax.numpy as jnp
from jax.experimental import pallas as pl


def kernel(nodes, X, neig, W1, W2, W, b):
    raise NotImplementedError("write your pallas kernel here")


# speedup vs baseline: 1.5170x; 1.5170x over previous
import functools

import jax
import jax.numpy as jnp
from jax import lax
from jax.experimental import pallas as pl
from jax.experimental.pallas import tpu as pltpu
from jax.experimental.pallas import tpu_sc as plsc

_LANES = 16


@functools.partial(jax.jit, static_argnames=("n_pad", "d", "m"))
def _sc_neighbor_sum(table, neig2d, *, n_pad, d, m):
    info = plsc.get_sparse_core_info()
    nw = info.num_cores * info.num_subcores
    g = 128 // m
    cn = n_pad // nw
    ngc = cn // g
    vregs_per_node = d // _LANES

    mesh = plsc.VectorSubcoreMesh(core_axis_name="c", subcore_axis_name="s")

    @functools.partial(
        pl.kernel,
        out_type=jax.ShapeDtypeStruct((n_pad, d), jnp.float32),
        mesh=mesh,
        scratch_types=[
            pltpu.VMEM((ngc, g * m), jnp.int32),
            pltpu.VMEM((2, g * m, d), jnp.float32),
            pltpu.VMEM((cn, d), jnp.float32),
            pltpu.SemaphoreType.DMA,
            pltpu.SemaphoreType.DMA,
        ],
    )
    def sc_kernel(table_hbm, neig_hbm, out_hbm, idx_v, rows_v, stage_v, sem0, sem1):
        wid = lax.axis_index("s") * info.num_cores + lax.axis_index("c")
        row0 = wid * ngc

        pltpu.sync_copy(neig_hbm.at[pl.ds(row0, ngc)], idx_v)

        sems = (sem0, sem1)
        pltpu.async_copy(table_hbm.at[idx_v.at[0]], rows_v.at[0], sem0)
        pltpu.async_copy(table_hbm.at[idx_v.at[1]], rows_v.at[1], sem1)

        @pl.loop(0, ngc, step=2)
        def _outer(c0):
            for b in range(2):
                c = c0 + b
                pltpu.make_async_copy(
                    table_hbm.at[idx_v.at[c]], rows_v.at[b], sems[b]
                ).wait()
                accs = [None] * (g * vregs_per_node)
                for n in range(g):
                    for j in range(m):
                        r = n * m + j
                        for f in range(vregs_per_node):
                            v = rows_v[b, r, pl.ds(f * _LANES, _LANES)]
                            a = accs[n * vregs_per_node + f]
                            accs[n * vregs_per_node + f] = v if a is None else a + v
                for n in range(g):
                    for f in range(vregs_per_node):
                        stage_v[c * g + n, pl.ds(f * _LANES, _LANES)] = (
                            accs[n * vregs_per_node + f]
                        )
                @pl.when(c + 2 < ngc)
                def _():
                    pltpu.async_copy(
                        table_hbm.at[idx_v.at[c + 2]], rows_v.at[b], sems[b]
                    )

        pltpu.sync_copy(stage_v, out_hbm.at[pl.ds(wid * cn, cn)])

    return sc_kernel(table, neig2d)


def _enc_body(x_ref, s_ref, wa_ref, wb_ref, o_ref):
    acc = jnp.dot(x_ref[...], wa_ref[...], preferred_element_type=jnp.float32)
    acc += jnp.dot(s_ref[...], wb_ref[...], preferred_element_type=jnp.float32)
    o_ref[...] = jnp.maximum(acc, 0.0)


def _head_body(x_ref, s_ref, wa_ref, wb_ref, w_ref, b_ref, o_ref):
    acc = jnp.dot(x_ref[...], wa_ref[...], preferred_element_type=jnp.float32)
    acc += jnp.dot(s_ref[...], wb_ref[...], preferred_element_type=jnp.float32)
    h = jnp.maximum(acc, 0.0)
    y = jnp.dot(h, w_ref[...], preferred_element_type=jnp.float32) + b_ref[...]
    ymax = jnp.max(y, axis=1, keepdims=True)
    z = y - ymax
    o_ref[...] = z - jnp.log(jnp.sum(jnp.exp(z), axis=1, keepdims=True))


def _tc_encoder(x, s, wa, wb, block_rows):
    n, d = x.shape
    h = wa.shape[1]
    nb = n // block_rows
    return pl.pallas_call(
        _enc_body,
        grid=(nb,),
        in_specs=[
            pl.BlockSpec((block_rows, d), lambda i: (i, 0)),
            pl.BlockSpec((block_rows, d), lambda i: (i, 0)),
            pl.BlockSpec((d, h), lambda i: (0, 0)),
            pl.BlockSpec((d, h), lambda i: (0, 0)),
        ],
        out_specs=pl.BlockSpec((block_rows, h), lambda i: (i, 0)),
        out_shape=jax.ShapeDtypeStruct((n, h), jnp.float32),
    )(x, s, wa, wb)


def _tc_head(x, s, wa, wb, w, b2d, block_rows):
    n, d = x.shape
    h = wa.shape[1]
    c = w.shape[1]
    nb = n // block_rows
    return pl.pallas_call(
        _head_body,
        grid=(nb,),
        in_specs=[
            pl.BlockSpec((block_rows, d), lambda i: (i, 0)),
            pl.BlockSpec((block_rows, d), lambda i: (i, 0)),
            pl.BlockSpec((d, h), lambda i: (0, 0)),
            pl.BlockSpec((d, h), lambda i: (0, 0)),
            pl.BlockSpec((h, c), lambda i: (0, 0)),
            pl.BlockSpec((1, c), lambda i: (0, 0)),
        ],
        out_specs=pl.BlockSpec((block_rows, c), lambda i: (i, 0)),
        out_shape=jax.ShapeDtypeStruct((n, c), jnp.float32),
    )(x, s, wa, wb, w, b2d)


def kernel(nodes, X, neig, W1, W2, W, b):
    n, d = X.shape
    m = neig.shape[1]
    nw = 32
    g = 128 // m
    cn = -(-n // nw)
    cn = -(-cn // 8) * 8
    n_pad = cn * nw

    neig_i = neig.astype(jnp.int32)
    neig_pad = jnp.pad(neig_i, ((0, n_pad - n), (0, 0)))
    neig2d = neig_pad.reshape(n_pad // g, g * m)

    inv_m = 1.0 / m
    w1a, w1b = W1[:d], W1[d:] * inv_m
    w2a, w2b = W2[:d], W2[d:] * inv_m
    b2d = b.reshape(1, -1)

    s1 = _sc_neighbor_sum(X, neig2d, n_pad=n_pad, d=d, m=m)[:n]
    h1 = _tc_encoder(X, s1, w1a, w1b, block_rows=2000)
    s2 = _sc_neighbor_sum(h1, neig2d, n_pad=n_pad, d=h1.shape[1], m=m)[:n]
    out = _tc_head(h1, s2, w2a, w2b, W, b2d, block_rows=2000)
    return jnp.take(out, nodes, axis=0)
